# baseline (speedup 1.0000x reference)
import functools

import jax
import jax.numpy as jnp
from jax import lax
from jax.experimental import pallas as pl
from jax.experimental.pallas import tpu as pltpu
from jax.experimental.pallas import tpu_sc as plsc

NCLS = 7
B_TOTAL = 1048576
UNC_THR = 0.5
DEC_THR = 0.25
SPEC_W = 0.7

_NC = 2
_NS = 16
_NW = _NC * _NS
_ROWS_PER_W = B_TOTAL // _NW
_CH = 4096
_N_CHUNKS = _ROWS_PER_W // _CH
_VECS = _CH // 16


def _body(probs_hbm, cu_hbm, sr_hbm, pr_hbm, out_hbm,
          pv0, pv1, cv0, cv1, sv0, sv1, rv0, rv1, ov0, ov1,
          isem, osem):
    wid = lax.axis_index("s") * _NC + lax.axis_index("c")
    w0 = wid * _ROWS_PER_W

    pv = (pv0, pv1)
    cv = (cv0, cv1)
    sv = (sv0, sv1)
    rv = (rv0, rv1)
    ov = (ov0, ov1)

    def in_copies(t, s):
        base = w0 + t * _CH
        sl = pl.ds(base, _CH)
        return (
            pltpu.make_async_copy(probs_hbm.at[pl.ds(0, NCLS), sl], pv[s], isem.at[s, 0]),
            pltpu.make_async_copy(cu_hbm.at[sl], cv[s], isem.at[s, 1]),
            pltpu.make_async_copy(sr_hbm.at[sl], sv[s], isem.at[s, 2]),
            pltpu.make_async_copy(pr_hbm.at[sl], rv[s], isem.at[s, 3]),
        )

    def out_copy(t, s):
        base = w0 + t * _CH
        return pltpu.make_async_copy(ov[s], out_hbm.at[pl.ds(base, _CH)], osem.at[s])

    def compute(s):
        pvs, cvs, svs, rvs, ovs = pv[s], cv[s], sv[s], rv[s], ov[s]

        def vec_body(j, _):
            r0 = j * 16
            sl = pl.ds(r0, 16)
            mv = pvs[0, sl]
            mi = jnp.zeros((16,), jnp.int32)
            for c in range(1, NCLS):
                g = pvs[c, sl]
                p = g > mv
                mv = jnp.where(p, g, mv)
                mi = jnp.where(p, c, mi)

            cu = cvs[sl]
            sr = svs[sl]
            pr = rvs[sl]

            dm = jnp.abs(sr - pr) > DEC_THR
            us = sr > pr
            spec = jnp.maximum(1.0 - sr, SPEC_W * (1.0 - pr))
            spat = jnp.maximum(1.0 - pr, SPEC_W * (1.0 - sr))
            fu = jnp.where(dm & us, spec, jnp.where(dm & (~us), spat, cu))
            rs = SPEC_W * fu + (1.0 - mv)
            unk = rs > UNC_THR
            ovs[sl] = jnp.where(unk, NCLS, mi)
            return 0

        lax.fori_loop(0, _VECS, vec_body, 0, unroll=4)

    for c in in_copies(0, 0):
        c.start()
    for t in range(_N_CHUNKS):
        s = t % 2
        if t + 1 < _N_CHUNKS:
            for c in in_copies(t + 1, (t + 1) % 2):
                c.start()
        for c in in_copies(t, s):
            c.wait()
        if t >= 2:
            out_copy(t - 2, s).wait()
        compute(s)
        out_copy(t, s).start()
    out_copy(_N_CHUNKS - 2, (_N_CHUNKS - 2) % 2).wait()
    out_copy(_N_CHUNKS - 1, (_N_CHUNKS - 1) % 2).wait()


_sc_call = functools.partial(
    pl.kernel,
    out_type=jax.ShapeDtypeStruct((B_TOTAL,), jnp.int32),
    mesh=plsc.VectorSubcoreMesh(core_axis_name="c", subcore_axis_name="s"),
    compiler_params=pltpu.CompilerParams(
        needs_layout_passes=False,
        use_tc_tiling_on_sc=True,
    ),
    scratch_types=(
        [pltpu.VMEM((NCLS, _CH), jnp.float32)] * 2
        + [pltpu.VMEM((_CH,), jnp.float32)] * 6
        + [pltpu.VMEM((_CH,), jnp.int32)] * 2
        + [pltpu.SemaphoreType.DMA((2, 4)), pltpu.SemaphoreType.DMA((2,))]
    ),
)(_body)


def kernel(probs, uncertainty_combined, spectral_reliability, spatial_reliability):
    return _sc_call(
        probs.T,
        uncertainty_combined.reshape(-1),
        spectral_reliability.reshape(-1),
        spatial_reliability.reshape(-1),
    )


# speedup vs baseline: 1.0362x; 1.0362x over previous
import functools

import jax
import jax.numpy as jnp
from jax import lax
from jax.experimental import pallas as pl
from jax.experimental.pallas import tpu as pltpu
from jax.experimental.pallas import tpu_sc as plsc

NCLS = 7
B_TOTAL = 1048576
UNC_THR = 0.5
DEC_THR = 0.25
SPEC_W = 0.7

_NC = 2
_NS = 16
_NW = _NC * _NS
_ROWS_PER_W = B_TOTAL // _NW
_CH = 2048
_N_CHUNKS = _ROWS_PER_W // _CH
_VECS = _CH // 16


def _body(probs_hbm, cu_hbm, sr_hbm, pr_hbm, out_hbm,
          pv0, pv1, cv0, cv1, sv0, sv1, rv0, rv1, ov0, ov1,
          isem, osem):
    wid = lax.axis_index("s") * _NC + lax.axis_index("c")
    w0 = wid * _ROWS_PER_W

    pv = (pv0, pv1)
    cv = (cv0, cv1)
    sv = (sv0, sv1)
    rv = (rv0, rv1)
    ov = (ov0, ov1)

    def in_copies(t, s):
        base = w0 + t * _CH
        sl = pl.ds(base, _CH)
        return (
            pltpu.make_async_copy(probs_hbm.at[pl.ds(0, NCLS), sl], pv[s], isem.at[s, 0]),
            pltpu.make_async_copy(cu_hbm.at[sl], cv[s], isem.at[s, 1]),
            pltpu.make_async_copy(sr_hbm.at[sl], sv[s], isem.at[s, 2]),
            pltpu.make_async_copy(pr_hbm.at[sl], rv[s], isem.at[s, 3]),
        )

    def out_copy(t, s):
        base = w0 + t * _CH
        return pltpu.make_async_copy(ov[s], out_hbm.at[pl.ds(base, _CH)], osem.at[s])

    def compute(s):
        pvs, cvs, svs, rvs, ovs = pv[s], cv[s], sv[s], rv[s], ov[s]

        def vec_body(j, _):
            r0 = j * 16
            sl = pl.ds(r0, 16)
            mv = pvs[0, sl]
            mi = jnp.zeros((16,), jnp.int32)
            for c in range(1, NCLS):
                g = pvs[c, sl]
                p = g > mv
                mv = jnp.where(p, g, mv)
                mi = jnp.where(p, c, mi)

            cu = cvs[sl]
            sr = svs[sl]
            pr = rvs[sl]

            dm = jnp.abs(sr - pr) > DEC_THR
            us = sr > pr
            spec = jnp.maximum(1.0 - sr, SPEC_W * (1.0 - pr))
            spat = jnp.maximum(1.0 - pr, SPEC_W * (1.0 - sr))
            fu = jnp.where(dm & us, spec, jnp.where(dm & (~us), spat, cu))
            rs = SPEC_W * fu + (1.0 - mv)
            unk = rs > UNC_THR
            ovs[sl] = jnp.where(unk, NCLS, mi)
            return 0

        lax.fori_loop(0, _VECS, vec_body, 0)

    for c in in_copies(0, 0):
        c.start()
    for t in range(_N_CHUNKS):
        s = t % 2
        if t + 1 < _N_CHUNKS:
            for c in in_copies(t + 1, (t + 1) % 2):
                c.start()
        for c in in_copies(t, s):
            c.wait()
        if t >= 2:
            out_copy(t - 2, s).wait()
        compute(s)
        out_copy(t, s).start()
    out_copy(_N_CHUNKS - 2, (_N_CHUNKS - 2) % 2).wait()
    out_copy(_N_CHUNKS - 1, (_N_CHUNKS - 1) % 2).wait()


_sc_call = functools.partial(
    pl.kernel,
    out_type=jax.ShapeDtypeStruct((B_TOTAL,), jnp.int32),
    mesh=plsc.VectorSubcoreMesh(core_axis_name="c", subcore_axis_name="s"),
    compiler_params=pltpu.CompilerParams(
        needs_layout_passes=False,
        use_tc_tiling_on_sc=True,
    ),
    scratch_types=(
        [pltpu.VMEM((NCLS, _CH), jnp.float32)] * 2
        + [pltpu.VMEM((_CH,), jnp.float32)] * 6
        + [pltpu.VMEM((_CH,), jnp.int32)] * 2
        + [pltpu.SemaphoreType.DMA((2, 4)), pltpu.SemaphoreType.DMA((2,))]
    ),
)(_body)


def kernel(probs, uncertainty_combined, spectral_reliability, spatial_reliability):
    return _sc_call(
        probs.T,
        uncertainty_combined.reshape(-1),
        spectral_reliability.reshape(-1),
        spatial_reliability.reshape(-1),
    )


# speedup vs baseline: 1.0510x; 1.0143x over previous
import functools

import jax
import jax.numpy as jnp
from jax import lax
from jax.experimental import pallas as pl
from jax.experimental.pallas import tpu as pltpu
from jax.experimental.pallas import tpu_sc as plsc

NCLS = 7
B_TOTAL = 1048576
UNC_THR = 0.5
DEC_THR = 0.25
SPEC_W = 0.7

_NC = 2
_NS = 16
_NW = _NC * _NS
_ROWS_PER_W = B_TOTAL // _NW
_CH = 4096
_N_CHUNKS = _ROWS_PER_W // _CH
_VECS = _CH // 16


def _body(probs_hbm, cu_hbm, sr_hbm, pr_hbm, out_hbm,
          pv0, pv1, cv0, cv1, sv0, sv1, rv0, rv1, ov0, ov1,
          isem, osem):
    wid = lax.axis_index("s") * _NC + lax.axis_index("c")
    w0 = wid * _ROWS_PER_W

    pv = (pv0, pv1)
    cv = (cv0, cv1)
    sv = (sv0, sv1)
    rv = (rv0, rv1)
    ov = (ov0, ov1)

    def in_copies(t, s):
        base = w0 + t * _CH
        sl = pl.ds(base, _CH)
        return (
            pltpu.make_async_copy(probs_hbm.at[pl.ds(0, NCLS), sl], pv[s], isem.at[s, 0]),
            pltpu.make_async_copy(cu_hbm.at[sl], cv[s], isem.at[s, 1]),
            pltpu.make_async_copy(sr_hbm.at[sl], sv[s], isem.at[s, 2]),
            pltpu.make_async_copy(pr_hbm.at[sl], rv[s], isem.at[s, 3]),
        )

    def out_copy(t, s):
        base = w0 + t * _CH
        return pltpu.make_async_copy(ov[s], out_hbm.at[pl.ds(base, _CH)], osem.at[s])

    def compute(s):
        pvs, cvs, svs, rvs, ovs = pv[s], cv[s], sv[s], rv[s], ov[s]

        def vec_body(j, _):
            r0 = j * 16
            sl = pl.ds(r0, 16)
            mv = pvs[0, sl]
            mi = jnp.zeros((16,), jnp.int32)
            for c in range(1, NCLS):
                g = pvs[c, sl]
                p = g > mv
                mv = jnp.where(p, g, mv)
                mi = jnp.where(p, c, mi)

            cu = cvs[sl]
            sr = svs[sl]
            pr = rvs[sl]

            dm = jnp.abs(sr - pr) > DEC_THR
            us = sr > pr
            spec = jnp.maximum(1.0 - sr, SPEC_W * (1.0 - pr))
            spat = jnp.maximum(1.0 - pr, SPEC_W * (1.0 - sr))
            fu = jnp.where(dm, jnp.where(us, spec, spat), cu)
            rs = SPEC_W * fu + (1.0 - mv)
            unk = rs > UNC_THR
            ovs[sl] = jnp.where(unk, NCLS, mi)
            return 0

        lax.fori_loop(0, _VECS, vec_body, 0)

    for c in in_copies(0, 0):
        c.start()
    for t in range(_N_CHUNKS):
        s = t % 2
        if t + 1 < _N_CHUNKS:
            for c in in_copies(t + 1, (t + 1) % 2):
                c.start()
        for c in in_copies(t, s):
            c.wait()
        if t >= 2:
            out_copy(t - 2, s).wait()
        compute(s)
        out_copy(t, s).start()
    out_copy(_N_CHUNKS - 2, (_N_CHUNKS - 2) % 2).wait()
    out_copy(_N_CHUNKS - 1, (_N_CHUNKS - 1) % 2).wait()


_sc_call = functools.partial(
    pl.kernel,
    out_type=jax.ShapeDtypeStruct((B_TOTAL,), jnp.int32),
    mesh=plsc.VectorSubcoreMesh(core_axis_name="c", subcore_axis_name="s"),
    compiler_params=pltpu.CompilerParams(
        needs_layout_passes=False,
        use_tc_tiling_on_sc=True,
    ),
    scratch_types=(
        [pltpu.VMEM((NCLS, _CH), jnp.float32)] * 2
        + [pltpu.VMEM((_CH,), jnp.float32)] * 6
        + [pltpu.VMEM((_CH,), jnp.int32)] * 2
        + [pltpu.SemaphoreType.DMA((2, 4)), pltpu.SemaphoreType.DMA((2,))]
    ),
)(_body)


def kernel(probs, uncertainty_combined, spectral_reliability, spatial_reliability):
    return _sc_call(
        probs.T,
        uncertainty_combined.reshape(-1),
        spectral_reliability.reshape(-1),
        spatial_reliability.reshape(-1),
    )
